# baseline (speedup 1.0000x reference)
import functools

import jax
import jax.numpy as jnp
from jax.experimental import pallas as pl
from jax.experimental.pallas import tpu as pltpu


def _pick_bm(n):
    for b in (400, 80, 16):
        if n % b == 0:
            return b
    return n


def _fused_kernel(feat_ref, feat_a_ref, w1_ref, adj_ref, wd_ref,
                  hid_ref, emb_ref, ret_ref, reta_ref,
                  z_sc, e_sc, *, np_):
    i = pl.program_id(0)
    h = w1_ref.shape[1]
    bm = adj_ref.shape[0]

    @pl.when(i == 0)
    def _prologue():
        w = w1_ref[...]
        z_sc[:, :h] = jnp.dot(feat_ref[...], w,
                              preferred_element_type=jnp.float32
                              ).astype(jnp.bfloat16)
        z_sc[:, h:] = jnp.dot(feat_a_ref[...], w,
                              preferred_element_type=jnp.float32
                              ).astype(jnp.bfloat16)

    @pl.when((i >= 1) & (i <= np_))
    def _pass1():
        m = i - 1
        acc = jnp.dot(adj_ref[...].astype(jnp.bfloat16), z_sc[...],
                      preferred_element_type=jnp.float32)
        hid_ref[...] = acc[:, :h]
        e = jnp.maximum(acc, 0.0)
        emb_ref[...] = e[:, :h]
        e_sc[pl.ds(m * bm, bm), :] = e.astype(jnp.bfloat16)

    @pl.when(i > np_)
    def _pass2():
        m = i - np_ - 1
        v = jnp.dot(adj_ref[...].astype(jnp.bfloat16), e_sc[...],
                    preferred_element_type=jnp.float32)
        v1 = v[:, :h]
        v2 = v[:, h:]
        n1 = jnp.sqrt(jnp.sum(v1 * v1, axis=1, keepdims=True))
        n2 = jnp.sqrt(jnp.sum(v2 * v2, axis=1, keepdims=True))
        g1 = jax.nn.sigmoid(v1 / jnp.maximum(n1, 1e-12))
        g2 = jax.nn.sigmoid(v2 / jnp.maximum(n2, 1e-12))
        wd = wd_ref[...]
        eb = e_sc[pl.ds(m * bm, bm), :]
        p1 = jnp.dot(eb[:, :h], wd, preferred_element_type=jnp.float32)
        p2 = jnp.dot(eb[:, h:], wd, preferred_element_type=jnp.float32)
        s11 = jnp.sum(p1 * g1, axis=1, keepdims=True)
        s21 = jnp.sum(p2 * g1, axis=1, keepdims=True)
        s22 = jnp.sum(p2 * g2, axis=1, keepdims=True)
        s12 = jnp.sum(p1 * g2, axis=1, keepdims=True)
        ret_ref[...] = jnp.concatenate([s11, s21], axis=1)
        reta_ref[...] = jnp.concatenate([s22, s12], axis=1)


def kernel(feat, feat_a, adj, weight1, weight2, disc_w, disc_b):
    n, f_in = feat.shape
    h = weight1.shape[1]
    bm = _pick_bm(n)
    np_ = n // bm

    def adj_idx(i):
        m1 = jnp.maximum(i - 1, 0)
        m2 = i - np_ - 1
        return (jnp.where(i > np_, m2, jnp.minimum(m1, np_ - 1)), 0)

    def p1_idx(i):
        return (jnp.clip(i - 1, 0, np_ - 1), 0)

    def p2_idx(i):
        return (jnp.clip(i - np_ - 1, 0, np_ - 1), 0)

    const_idx = lambda i: (0, 0)

    fb = feat.astype(jnp.bfloat16)
    fab = feat_a.astype(jnp.bfloat16)
    w1b = weight1.astype(jnp.bfloat16)
    wdb = disc_w.reshape(h, h).astype(jnp.bfloat16)

    hid, emb, retr, reta = pl.pallas_call(
        functools.partial(_fused_kernel, np_=np_),
        grid=(2 * np_ + 1,),
        in_specs=[
            pl.BlockSpec((n, f_in), const_idx),
            pl.BlockSpec((n, f_in), const_idx),
            pl.BlockSpec((f_in, h), const_idx),
            pl.BlockSpec((bm, n), adj_idx),
            pl.BlockSpec((h, h), const_idx),
        ],
        out_specs=[
            pl.BlockSpec((bm, h), p1_idx),
            pl.BlockSpec((bm, h), p1_idx),
            pl.BlockSpec((bm, 2), p2_idx),
            pl.BlockSpec((bm, 2), p2_idx),
        ],
        out_shape=[
            jax.ShapeDtypeStruct((n, h), jnp.float32),
            jax.ShapeDtypeStruct((n, h), jnp.float32),
            jax.ShapeDtypeStruct((n, 2), jnp.float32),
            jax.ShapeDtypeStruct((n, 2), jnp.float32),
        ],
        scratch_shapes=[
            pltpu.VMEM((n, 2 * h), jnp.bfloat16),
            pltpu.VMEM((n, 2 * h), jnp.bfloat16),
        ],
        compiler_params=pltpu.CompilerParams(
            dimension_semantics=("arbitrary",),
            vmem_limit_bytes=60 * 1024 * 1024,
        ),
    )(fb, fab, w1b, adj, wdb)

    b0 = disc_b[0]
    return hid, emb, retr + b0, reta + b0
